# Initial kernel scaffold; baseline (speedup 1.0000x reference)
import jax
import jax.numpy as jnp
from jax.experimental import pallas as pl


def kernel(x, wg, w1, w2, w3, sw1, sw2, sw3):
    raise NotImplementedError("write your pallas kernel here")


# speedup vs baseline: 1.5158x; 1.5158x over previous
import functools

import jax
import jax.numpy as jnp
from jax import lax
from jax.experimental import pallas as pl
from jax.experimental.pallas import tpu as pltpu
from jax.experimental.pallas import tpu_sc as plsc

BS, SLEN, DIM = 2, 2048, 2048
HID = 2048
E = 8
K = 2
T = BS * SLEN
TK = T * K

BM = 128
BH = 512
M_ROUTED = TK + E * BM
TILES_R = M_ROUTED // BM
TILES_S = T // BM
HB = HID // BH

NW = 32


def _router_body(x_ref, wg_ref, o_ref):
    s = lax.dot_general(x_ref[...], wg_ref[...], (((1,), (1,)), ((), ())),
                        preferred_element_type=jnp.float32)
    o_ref[...] = jax.nn.sigmoid(s)


def _router(xf, wg):
    bm = 512
    return pl.pallas_call(
        _router_body,
        grid=(T // bm,),
        in_specs=[
            pl.BlockSpec((bm, DIM), lambda i: (i, 0)),
            pl.BlockSpec((E, DIM), lambda i: (0, 0)),
        ],
        out_specs=pl.BlockSpec((bm, E), lambda i: (i, 0)),
        out_shape=jax.ShapeDtypeStruct((T, E), jnp.float32),
    )(xf, wg)


def _ffn_body(eids_ref, x_ref, s_ref, w1_ref, w3_ref, w2_ref, o_ref):
    j = pl.program_id(1)
    x = x_ref[...] * s_ref[...]
    a = lax.dot_general(x, w1_ref[0], (((1,), (1,)), ((), ())),
                        preferred_element_type=jnp.float32)
    b = lax.dot_general(x, w3_ref[0], (((1,), (1,)), ((), ())),
                        preferred_element_type=jnp.float32)
    h = (a * jax.nn.sigmoid(a)) * b
    y = lax.dot_general(h, w2_ref[0], (((1,), (1,)), ((), ())),
                        preferred_element_type=jnp.float32)

    @pl.when(j == 0)
    def _():
        o_ref[...] = y

    @pl.when(j > 0)
    def _():
        o_ref[...] += y


def _grouped_ffn(xbuf, scales, w1e, w3e, w2e, eids, n_tiles):
    n = n_tiles * BM
    grid_spec = pltpu.PrefetchScalarGridSpec(
        num_scalar_prefetch=1,
        grid=(n_tiles, HB),
        in_specs=[
            pl.BlockSpec((BM, DIM), lambda i, j, eids: (i, 0)),
            pl.BlockSpec((BM, 1), lambda i, j, eids: (i, 0)),
            pl.BlockSpec((1, BH, DIM), lambda i, j, eids: (eids[i], j, 0)),
            pl.BlockSpec((1, BH, DIM), lambda i, j, eids: (eids[i], j, 0)),
            pl.BlockSpec((1, DIM, BH), lambda i, j, eids: (eids[i], 0, j)),
        ],
        out_specs=pl.BlockSpec((BM, DIM), lambda i, j, eids: (i, 0)),
    )
    return pl.pallas_call(
        _ffn_body,
        grid_spec=grid_spec,
        out_shape=jax.ShapeDtypeStruct((n, DIM), jnp.float32),
        compiler_params=pltpu.CompilerParams(
            dimension_semantics=("arbitrary", "arbitrary")),
    )(eids, xbuf, scales, w1e, w3e, w2e)


def _add3_body(a_ref, b_ref, c_ref, o_ref):
    o_ref[...] = a_ref[...] + b_ref[...] + c_ref[...]


def _add3(y01, ys):
    bm = 256
    nb = T // bm
    return pl.pallas_call(
        _add3_body,
        grid=(nb,),
        in_specs=[
            pl.BlockSpec((bm, DIM), lambda i: (i, 0)),
            pl.BlockSpec((bm, DIM), lambda i, nb=nb: (i + nb, 0)),
            pl.BlockSpec((bm, DIM), lambda i: (i, 0)),
        ],
        out_specs=pl.BlockSpec((bm, DIM), lambda i: (i, 0)),
        out_shape=jax.ShapeDtypeStruct((T, DIM), jnp.float32),
    )(y01, y01, ys)


@functools.lru_cache(maxsize=None)
def _make_gather(n_rows, chunk):
    per_w = n_rows // NW
    n_chunks = per_w // chunk
    assert per_w % chunk == 0 and per_w % 8 == 0 and chunk % 8 == 0
    mesh = plsc.VectorSubcoreMesh(core_axis_name="c", subcore_axis_name="s",
                                  num_cores=2, num_subcores=16)

    @functools.partial(
        pl.kernel,
        out_type=jax.ShapeDtypeStruct((n_rows, DIM), jnp.float32),
        mesh=mesh,
        scratch_types=[
            pltpu.VMEM((chunk,), jnp.int32),
            pltpu.VMEM((chunk, DIM), jnp.float32),
            pltpu.SemaphoreType.DMA,
        ],
    )
    def gk(table_hbm, idx_hbm, out_hbm, idx_v, rows_v, sem):
        wid = lax.axis_index("s") * 2 + lax.axis_index("c")
        for c in range(n_chunks):
            base = wid * per_w + c * chunk
            pltpu.sync_copy(idx_hbm.at[pl.ds(base, chunk)], idx_v)
            pltpu.async_copy(table_hbm.at[idx_v], rows_v, sem).wait()
            pltpu.sync_copy(rows_v, out_hbm.at[pl.ds(base, chunk)])

    return gk


def _metadata(scores):
    top_scores, sel = lax.top_k(scores, K)
    flat_e = sel.reshape(-1).astype(jnp.int32)
    order = jnp.argsort(flat_e, stable=True).astype(jnp.int32)
    counts = jnp.zeros((E,), jnp.int32).at[flat_e].add(1)
    padded = ((counts + BM - 1) // BM) * BM
    pad_end = jnp.cumsum(padded)
    pad_start = pad_end - padded
    start = jnp.cumsum(counts) - counts
    sorted_e = flat_e[order]
    pos_sorted = (pad_start[sorted_e]
                  + jnp.arange(TK, dtype=jnp.int32) - start[sorted_e])
    src_tok = jnp.zeros((M_ROUTED,), jnp.int32).at[pos_sorted].set(order // K)
    scale = jnp.zeros((M_ROUTED,), jnp.float32).at[pos_sorted].set(
        top_scores.reshape(-1)[order])
    r_pair = jnp.zeros((TK,), jnp.int32).at[order].set(pos_sorted)
    r01 = jnp.concatenate([r_pair[0::K], r_pair[1::K]])
    tile_base = jnp.arange(TILES_R, dtype=jnp.int32) * BM
    eids = jnp.sum((tile_base[:, None] >= pad_end[None, :]).astype(jnp.int32),
                   axis=1)
    eids = jnp.minimum(eids, E - 1)
    return src_tok, scale, r01, eids


def kernel(x, wg, w1, w2, w3, sw1, sw2, sw3):
    xf = x.reshape(-1, DIM)
    scores = _router(xf, wg)
    src_tok, scale, r01, eids = _metadata(scores)

    xbuf = _make_gather(M_ROUTED, 24)(xf, src_tok)
    yr = _grouped_ffn(xbuf, scale[:, None], w1, w3, w2, eids, TILES_R)

    ones = jnp.ones((T, 1), jnp.float32)
    zeids = jnp.zeros((TILES_S,), jnp.int32)
    ys = _grouped_ffn(xf, ones, sw1[None], sw3[None], sw2[None], zeids,
                      TILES_S)

    y01 = _make_gather(TK, 32)(yr, r01)
    out = _add3(y01, ys)
    return out.reshape(x.shape)


# speedup vs baseline: 2.0003x; 1.3196x over previous
import functools

import jax
import jax.numpy as jnp
from jax import lax
from jax.experimental import pallas as pl
from jax.experimental.pallas import tpu as pltpu
from jax.experimental.pallas import tpu_sc as plsc

BS, SLEN, DIM = 2, 2048, 2048
HID = 2048
E = 8
K = 2
T = BS * SLEN
TK = T * K

BM = 128
M_ROUTED = TK + E * BM
TILES_R = M_ROUTED // BM
TILES_S = T // BM

NW = 32


def _router_body(x_ref, wg_ref, o_ref):
    s = lax.dot_general(x_ref[...], wg_ref[...], (((1,), (1,)), ((), ())),
                        preferred_element_type=jnp.float32)
    o_ref[...] = jax.nn.sigmoid(s)


def _router(xf, wg):
    bm = 512
    return pl.pallas_call(
        _router_body,
        grid=(T // bm,),
        in_specs=[
            pl.BlockSpec((bm, DIM), lambda i: (i, 0)),
            pl.BlockSpec((E, DIM), lambda i: (0, 0)),
        ],
        out_specs=pl.BlockSpec((bm, E), lambda i: (i, 0)),
        out_shape=jax.ShapeDtypeStruct((T, E), jnp.float32),
    )(xf, wg)


def _ffn_body(eids_ref, x_ref, s_ref, w1_ref, w3_ref, w2_ref, o_ref):
    x = (x_ref[...] * s_ref[...]).astype(jnp.bfloat16)
    a = lax.dot_general(x, w1_ref[0], (((1,), (1,)), ((), ())),
                        preferred_element_type=jnp.float32)
    b = lax.dot_general(x, w3_ref[0], (((1,), (1,)), ((), ())),
                        preferred_element_type=jnp.float32)
    h = ((a * jax.nn.sigmoid(a)) * b).astype(jnp.bfloat16)
    o_ref[...] = lax.dot_general(h, w2_ref[0], (((1,), (1,)), ((), ())),
                                 preferred_element_type=jnp.float32)


def _grouped_ffn(xbuf, scales, w1e, w3e, w2e, eids, n_tiles):
    n = n_tiles * BM
    grid_spec = pltpu.PrefetchScalarGridSpec(
        num_scalar_prefetch=1,
        grid=(n_tiles,),
        in_specs=[
            pl.BlockSpec((BM, DIM), lambda i, eids: (i, 0)),
            pl.BlockSpec((BM, 1), lambda i, eids: (i, 0)),
            pl.BlockSpec((1, HID, DIM), lambda i, eids: (eids[i], 0, 0)),
            pl.BlockSpec((1, HID, DIM), lambda i, eids: (eids[i], 0, 0)),
            pl.BlockSpec((1, DIM, HID), lambda i, eids: (eids[i], 0, 0)),
        ],
        out_specs=pl.BlockSpec((BM, DIM), lambda i, eids: (i, 0)),
    )
    return pl.pallas_call(
        _ffn_body,
        grid_spec=grid_spec,
        out_shape=jax.ShapeDtypeStruct((n, DIM), jnp.float32),
        compiler_params=pltpu.CompilerParams(
            dimension_semantics=("arbitrary",)),
    )(eids, xbuf, scales, w1e, w3e, w2e)


def _add3_body(a_ref, b_ref, c_ref, o_ref):
    o_ref[...] = a_ref[...] + b_ref[...] + c_ref[...]


def _add3(y01, ys):
    bm = 256
    nb = T // bm
    return pl.pallas_call(
        _add3_body,
        grid=(nb,),
        in_specs=[
            pl.BlockSpec((bm, DIM), lambda i: (i, 0)),
            pl.BlockSpec((bm, DIM), lambda i, nb=nb: (i + nb, 0)),
            pl.BlockSpec((bm, DIM), lambda i: (i, 0)),
        ],
        out_specs=pl.BlockSpec((bm, DIM), lambda i: (i, 0)),
        out_shape=jax.ShapeDtypeStruct((T, DIM), jnp.float32),
    )(y01, y01, ys)


@functools.lru_cache(maxsize=None)
def _make_gather(n_rows, chunk):
    per_w = n_rows // NW
    n_chunks = per_w // chunk
    assert per_w % chunk == 0 and per_w % 8 == 0 and chunk % 8 == 0
    mesh = plsc.VectorSubcoreMesh(core_axis_name="c", subcore_axis_name="s",
                                  num_cores=2, num_subcores=16)

    @functools.partial(
        pl.kernel,
        out_type=jax.ShapeDtypeStruct((n_rows, DIM), jnp.float32),
        mesh=mesh,
        scratch_types=[
            pltpu.VMEM((chunk,), jnp.int32),
            pltpu.VMEM((chunk, DIM), jnp.float32),
            pltpu.SemaphoreType.DMA,
        ],
    )
    def gk(table_hbm, idx_hbm, out_hbm, idx_v, rows_v, sem):
        wid = lax.axis_index("s") * 2 + lax.axis_index("c")
        for c in range(n_chunks):
            base = wid * per_w + c * chunk
            pltpu.sync_copy(idx_hbm.at[pl.ds(base, chunk)], idx_v)
            pltpu.async_copy(table_hbm.at[idx_v], rows_v, sem).wait()
            pltpu.sync_copy(rows_v, out_hbm.at[pl.ds(base, chunk)])

    return gk


def _metadata(scores):
    top_scores, sel = lax.top_k(scores, K)
    flat_e = sel.reshape(-1).astype(jnp.int32)
    onehot = (flat_e[:, None] == jnp.arange(E, dtype=jnp.int32)[None, :])
    csum = jnp.cumsum(onehot.astype(jnp.int32), axis=0)
    counts = csum[-1]
    padded = ((counts + BM - 1) // BM) * BM
    pad_end = jnp.cumsum(padded)
    pad_start = pad_end - padded
    rank = jnp.take_along_axis(csum, flat_e[:, None], axis=1)[:, 0] - 1
    r_pair = pad_start[flat_e] + rank
    src_tok = jnp.zeros((M_ROUTED,), jnp.int32).at[r_pair].set(
        jnp.arange(TK, dtype=jnp.int32) // K)
    scale = jnp.zeros((M_ROUTED,), jnp.float32).at[r_pair].set(
        top_scores.reshape(-1))
    r01 = jnp.concatenate([r_pair[0::K], r_pair[1::K]])
    tile_base = jnp.arange(TILES_R, dtype=jnp.int32) * BM
    eids = jnp.sum((tile_base[:, None] >= pad_end[None, :]).astype(jnp.int32),
                   axis=1)
    eids = jnp.minimum(eids, E - 1)
    return src_tok, scale, r01, eids


def kernel(x, wg, w1, w2, w3, sw1, sw2, sw3):
    xf = x.reshape(-1, DIM)
    scores = _router(xf, wg)
    src_tok, scale, r01, eids = _metadata(scores)

    w1b = w1.astype(jnp.bfloat16)
    w3b = w3.astype(jnp.bfloat16)
    w2b = w2.astype(jnp.bfloat16)

    xbuf = _make_gather(M_ROUTED, 24)(xf, src_tok)
    yr = _grouped_ffn(xbuf, scale[:, None], w1b, w3b, w2b, eids, TILES_R)

    ones = jnp.ones((T, 1), jnp.float32)
    zeids = jnp.zeros((TILES_S,), jnp.int32)
    ys = _grouped_ffn(xf, ones, sw1[None].astype(jnp.bfloat16),
                      sw3[None].astype(jnp.bfloat16),
                      sw2[None].astype(jnp.bfloat16), zeids, TILES_S)

    y01 = _make_gather(TK, 32)(yr, r01)
    out = _add3(y01, ys)
    return out.reshape(x.shape)


# speedup vs baseline: 2.1914x; 1.0956x over previous
import functools

import jax
import jax.numpy as jnp
from jax import lax
from jax.experimental import pallas as pl
from jax.experimental.pallas import tpu as pltpu
from jax.experimental.pallas import tpu_sc as plsc

BS, SLEN, DIM = 2, 2048, 2048
HID = 2048
E = 8
K = 2
T = BS * SLEN
TK = T * K

BM = 128
M_ROUTED = TK + E * BM
TILES_R = M_ROUTED // BM
TILES_S = T // BM

NW = 32


def _router_body(x_ref, wg_ref, o_ref):
    s = lax.dot_general(x_ref[...], wg_ref[...], (((1,), (1,)), ((), ())),
                        preferred_element_type=jnp.float32)
    o_ref[...] = jax.nn.sigmoid(s)


def _router(xf, wg):
    bm = 512
    return pl.pallas_call(
        _router_body,
        grid=(T // bm,),
        in_specs=[
            pl.BlockSpec((bm, DIM), lambda i: (i, 0)),
            pl.BlockSpec((E, DIM), lambda i: (0, 0)),
        ],
        out_specs=pl.BlockSpec((bm, E), lambda i: (i, 0)),
        out_shape=jax.ShapeDtypeStruct((T, E), jnp.float32),
    )(xf, wg)


def _ffn_body(eids_ref, first_ref,
              x_ref, s_ref, w1_hbm, w3_hbm, w2_hbm, o_ref,
              w1v, w3v, w2v, wsem):
    i = pl.program_id(0)
    e = eids_ref[i]
    first = first_ref[i] == 1

    @pl.when(first)
    def _():
        pltpu.make_async_copy(w1_hbm.at[e], w1v, wsem.at[0]).start()
        pltpu.make_async_copy(w3_hbm.at[e], w3v, wsem.at[1]).start()
        pltpu.make_async_copy(w2_hbm.at[e], w2v, wsem.at[2]).start()

    @pl.when(first)
    def _():
        pltpu.make_async_copy(w1_hbm.at[e], w1v, wsem.at[0]).wait()

    x = x_ref[...] * s_ref[...]
    a = lax.dot_general(x, w1v[...], (((1,), (1,)), ((), ())),
                        preferred_element_type=jnp.float32)

    @pl.when(first)
    def _():
        pltpu.make_async_copy(w3_hbm.at[e], w3v, wsem.at[1]).wait()

    b = lax.dot_general(x, w3v[...], (((1,), (1,)), ((), ())),
                        preferred_element_type=jnp.float32)
    h = (a * jax.nn.sigmoid(a)) * b

    @pl.when(first)
    def _():
        pltpu.make_async_copy(w2_hbm.at[e], w2v, wsem.at[2]).wait()

    o_ref[...] = lax.dot_general(h, w2v[...], (((1,), (1,)), ((), ())),
                                 preferred_element_type=jnp.float32)


def _run_meta(eids):
    n = eids.shape[0]
    idx = jnp.arange(n, dtype=jnp.int32)
    return jnp.where(idx == 0, 1,
                     (eids != jnp.roll(eids, 1)).astype(jnp.int32))


def _grouped_ffn(xbuf, scales, w1e, w3e, w2e, eids, n_tiles):
    n = n_tiles * BM
    first = _run_meta(eids)
    grid_spec = pltpu.PrefetchScalarGridSpec(
        num_scalar_prefetch=2,
        grid=(n_tiles,),
        in_specs=[
            pl.BlockSpec((BM, DIM), lambda i, *_: (i, 0)),
            pl.BlockSpec((BM, 1), lambda i, *_: (i, 0)),
            pl.BlockSpec(memory_space=pltpu.MemorySpace.HBM),
            pl.BlockSpec(memory_space=pltpu.MemorySpace.HBM),
            pl.BlockSpec(memory_space=pltpu.MemorySpace.HBM),
        ],
        out_specs=pl.BlockSpec((BM, DIM), lambda i, *_: (i, 0)),
        scratch_shapes=[
            pltpu.VMEM((HID, DIM), jnp.float32),
            pltpu.VMEM((HID, DIM), jnp.float32),
            pltpu.VMEM((DIM, HID), jnp.float32),
            pltpu.SemaphoreType.DMA((3,)),
        ],
    )
    return pl.pallas_call(
        _ffn_body,
        grid_spec=grid_spec,
        out_shape=jax.ShapeDtypeStruct((n, DIM), jnp.float32),
        compiler_params=pltpu.CompilerParams(
            dimension_semantics=("arbitrary",),
            vmem_limit_bytes=63 * 1024 * 1024),
    )(eids, first, xbuf, scales, w1e, w3e, w2e)


def _add3_body(a_ref, b_ref, c_ref, o_ref):
    o_ref[...] = a_ref[...] + b_ref[...] + c_ref[...]


def _add3(y01, ys):
    bm = 256
    nb = T // bm
    return pl.pallas_call(
        _add3_body,
        grid=(nb,),
        in_specs=[
            pl.BlockSpec((bm, DIM), lambda i: (i, 0)),
            pl.BlockSpec((bm, DIM), lambda i, nb=nb: (i + nb, 0)),
            pl.BlockSpec((bm, DIM), lambda i: (i, 0)),
        ],
        out_specs=pl.BlockSpec((bm, DIM), lambda i: (i, 0)),
        out_shape=jax.ShapeDtypeStruct((T, DIM), jnp.float32),
    )(y01, y01, ys)


@functools.lru_cache(maxsize=None)
def _make_gather(n_rows, chunk):
    per_w = n_rows // NW
    n_chunks = per_w // chunk
    assert per_w % chunk == 0 and per_w % 8 == 0 and chunk % 8 == 0
    mesh = plsc.VectorSubcoreMesh(core_axis_name="c", subcore_axis_name="s",
                                  num_cores=2, num_subcores=16)

    @functools.partial(
        pl.kernel,
        out_type=jax.ShapeDtypeStruct((n_rows, DIM), jnp.float32),
        mesh=mesh,
        scratch_types=[
            pltpu.VMEM((chunk,), jnp.int32),
            pltpu.VMEM((chunk, DIM), jnp.float32),
            pltpu.SemaphoreType.DMA,
        ],
    )
    def gk(table_hbm, idx_hbm, out_hbm, idx_v, rows_v, sem):
        wid = lax.axis_index("s") * 2 + lax.axis_index("c")
        for c in range(n_chunks):
            base = wid * per_w + c * chunk
            pltpu.sync_copy(idx_hbm.at[pl.ds(base, chunk)], idx_v)
            pltpu.async_copy(table_hbm.at[idx_v], rows_v, sem).wait()
            pltpu.sync_copy(rows_v, out_hbm.at[pl.ds(base, chunk)])

    return gk


def _metadata(scores):
    top_scores, sel = lax.top_k(scores, K)
    flat_e = sel.reshape(-1).astype(jnp.int32)
    onehot = (flat_e[:, None] == jnp.arange(E, dtype=jnp.int32)[None, :])
    csum = jnp.cumsum(onehot.astype(jnp.int32), axis=0)
    counts = csum[-1]
    padded = ((counts + BM - 1) // BM) * BM
    pad_end = jnp.cumsum(padded)
    pad_start = pad_end - padded
    rank = jnp.take_along_axis(csum, flat_e[:, None], axis=1)[:, 0] - 1
    r_pair = pad_start[flat_e] + rank
    src_tok = jnp.zeros((M_ROUTED,), jnp.int32).at[r_pair].set(
        jnp.arange(TK, dtype=jnp.int32) // K)
    scale = jnp.zeros((M_ROUTED,), jnp.float32).at[r_pair].set(
        top_scores.reshape(-1))
    r01 = jnp.concatenate([r_pair[0::K], r_pair[1::K]])
    tile_base = jnp.arange(TILES_R, dtype=jnp.int32) * BM
    eids = jnp.sum((tile_base[:, None] >= pad_end[None, :]).astype(jnp.int32),
                   axis=1)
    eids = jnp.minimum(eids, E - 1)
    return src_tok, scale, r01, eids


def kernel(x, wg, w1, w2, w3, sw1, sw2, sw3):
    xf = x.reshape(-1, DIM)
    scores = _router(xf, wg)
    src_tok, scale, r01, eids = _metadata(scores)

    xbuf = _make_gather(M_ROUTED, 24)(xf, src_tok)
    yr = _grouped_ffn(xbuf, scale[:, None], w1, w3, w2, eids, TILES_R)

    ones = jnp.ones((T, 1), jnp.float32)
    zeids = jnp.zeros((TILES_S,), jnp.int32)
    ys = _grouped_ffn(xf, ones, sw1[None], sw3[None], sw2[None], zeids,
                      TILES_S)

    y01 = _make_gather(TK, 32)(yr, r01)
    out = _add3(y01, ys)
    return out.reshape(x.shape)
